# Initial kernel scaffold; baseline (speedup 1.0000x reference)
import jax
import jax.numpy as jnp
from jax.experimental import pallas as pl


def kernel(e0, e1, e2, ent_embed, rule_embed, W, b):
    raise NotImplementedError("write your pallas kernel here")


# speedup vs baseline: 10.5760x; 10.5760x over previous
import functools

import jax
import jax.numpy as jnp
from jax import lax
from jax.experimental import pallas as pl
from jax.experimental.pallas import tpu as pltpu
from jax.experimental.pallas import tpu_sc as plsc

B = 1024
DIM = 128
R = 2
NBR = 16

NW = 32
N0 = B * R // NW
N1 = B * R * NBR // NW
N2 = B * R * NBR * NBR // NW
CHUNK = 128
NCH2 = N2 // CHUNK
OUT_PER_CHUNK = CHUNK // NBR


def _sc_body(e0_h, e1_h, e2_h, tab_h, v0_h, v1_h, s2_h,
             idx_v, buf_a, buf_b, stage, sem_a, sem_b):
    wid = lax.axis_index("s") * 2 + lax.axis_index("c")

    pltpu.sync_copy(e0_h.at[pl.ds(wid * N0, N0)], idx_v.at[pl.ds(0, N0)])
    pltpu.async_copy(tab_h.at[idx_v.at[pl.ds(0, N0)]],
                     buf_a.at[pl.ds(0, N0)], sem_a).wait()
    pltpu.sync_copy(buf_a.at[pl.ds(0, N0)], v0_h.at[pl.ds(wid * N0, N0)])

    pltpu.sync_copy(e1_h.at[pl.ds(wid * N1, N1)], idx_v.at[pl.ds(0, N1)])
    n1ch = N1 // CHUNK
    pend = [
        pltpu.async_copy(tab_h.at[idx_v.at[pl.ds(0, CHUNK)]], buf_a, sem_a),
        pltpu.async_copy(tab_h.at[idx_v.at[pl.ds(CHUNK, CHUNK)]], buf_b, sem_b),
    ]
    for j in range(n1ch):
        pend[j % 2].wait()
        buf = buf_a if j % 2 == 0 else buf_b
        sem = sem_a if j % 2 == 0 else sem_b
        pltpu.sync_copy(buf, v1_h.at[pl.ds(wid * N1 + j * CHUNK, CHUNK)])
        if j + 2 < n1ch:
            pend[j % 2] = pltpu.async_copy(
                tab_h.at[idx_v.at[pl.ds((j + 2) * CHUNK, CHUNK)]], buf, sem)

    pltpu.sync_copy(e2_h.at[pl.ds(wid * N2, N2)], idx_v)

    pltpu.async_copy(tab_h.at[idx_v.at[pl.ds(0, CHUNK)]], buf_a, sem_a)
    pltpu.async_copy(tab_h.at[idx_v.at[pl.ds(CHUNK, CHUNK)]], buf_b, sem_b)

    zero8 = tuple(jnp.zeros((16,), jnp.float32) for _ in range(8))

    def do_chunk(c, buf, sem):
        pltpu.make_async_copy(tab_h.at[pl.ds(0, CHUNK)], buf, sem).wait()
        for o in range(OUT_PER_CHUNK):
            base = o * NBR

            def rbody(r, accs, _base=base, _buf=buf):
                row = _base + r * 4
                for u in range(4):
                    accs = tuple(accs[k] + _buf[row + u, pl.ds(k * 16, 16)]
                                 for k in range(8))
                return accs

            accs = lax.fori_loop(0, NBR // 4, rbody, zero8)
            for k in range(8):
                stage[o, pl.ds(k * 16, 16)] = accs[k]
        pltpu.sync_copy(
            stage, s2_h.at[pl.ds(wid * (N2 // NBR) + c * OUT_PER_CHUNK,
                                 OUT_PER_CHUNK)])
        @pl.when(c + 2 < NCH2)
        def _():
            pltpu.async_copy(tab_h.at[idx_v.at[pl.ds((c + 2) * CHUNK, CHUNK)]],
                             buf, sem)

    def pair_body(g, _):
        do_chunk(2 * g, buf_a, sem_a)
        do_chunk(2 * g + 1, buf_b, sem_b)
        return 0

    lax.fori_loop(0, NCH2 // 2, pair_body, 0)


def _sc_gather(e0f, e1f, e2f, table):
    mesh = plsc.VectorSubcoreMesh(core_axis_name="c", subcore_axis_name="s")
    f = pl.kernel(
        _sc_body,
        out_type=[
            jax.ShapeDtypeStruct((B * R, DIM), jnp.float32),
            jax.ShapeDtypeStruct((B * R * NBR, DIM), jnp.float32),
            jax.ShapeDtypeStruct((B * R * NBR, DIM), jnp.float32),
        ],
        mesh=mesh,
        scratch_types=[
            pltpu.VMEM((N2,), jnp.int32),
            pltpu.VMEM((CHUNK, DIM), jnp.float32),
            pltpu.VMEM((CHUNK, DIM), jnp.float32),
            pltpu.VMEM((OUT_PER_CHUNK, DIM), jnp.float32),
            pltpu.SemaphoreType.DMA,
            pltpu.SemaphoreType.DMA,
        ],
    )
    return f(e0f, e1f, e2f, table)


def _tc_body(re_ref, v0_ref, v1_ref, s2_ref, wt_ref, b_ref, out_ref):
    u = v1_ref.shape[0] // (R * NBR)
    v1 = v1_ref[...]
    s2 = s2_ref[...]
    wt = wt_ref[...]
    bb = b_ref[...]
    prec = lax.Precision.HIGHEST

    s1 = v1 + s2 * (1.0 / NBR)
    h1 = jnp.maximum(jnp.dot(s1, wt, precision=prec) + bb, 0.0)

    agg1 = jnp.sum(v1.reshape(u * R, NBR, DIM), axis=1) * (1.0 / NBR)
    h0 = jnp.maximum(jnp.dot(v0_ref[...] + agg1, wt, precision=prec) + bb, 0.0)

    aggh1 = jnp.sum(h1.reshape(u * R, NBR, DIM), axis=1) * (1.0 / NBR)
    o = jnp.tanh(jnp.dot(h0 + aggh1, wt, precision=prec) + bb)

    o3 = o.reshape(u, R, DIM)
    r0 = re_ref[0, 0]
    r1 = re_ref[0, 1]
    out_ref[...] = o3[:, 0, :] * r0 + o3[:, 1, :] * r1


def _tc_dense(v0, v1, s2, wt, b2, re):
    grid = 8
    u = B // grid
    return pl.pallas_call(
        _tc_body,
        grid=(grid,),
        in_specs=[
            pl.BlockSpec(memory_space=pltpu.SMEM),
            pl.BlockSpec((u * R, DIM), lambda i: (i, 0)),
            pl.BlockSpec((u * R * NBR, DIM), lambda i: (i, 0)),
            pl.BlockSpec((u * R * NBR, DIM), lambda i: (i, 0)),
            pl.BlockSpec((DIM, DIM), lambda i: (0, 0)),
            pl.BlockSpec((1, DIM), lambda i: (0, 0)),
        ],
        out_specs=pl.BlockSpec((u, DIM), lambda i: (i, 0)),
        out_shape=jax.ShapeDtypeStruct((B, DIM), jnp.float32),
    )(re, v0, v1, s2, wt, b2)


def kernel(e0, e1, e2, ent_embed, rule_embed, W, b):
    v0, v1, s2 = _sc_gather(e0.reshape(-1), e1.reshape(-1), e2.reshape(-1),
                            ent_embed)
    return _tc_dense(v0, v1, s2, W.T, b.reshape(1, DIM), rule_embed)
